# baseline (speedup 1.0000x reference)
import functools

import jax
import jax.numpy as jnp
from jax import lax
from jax.experimental import pallas as pl
from jax.experimental.pallas import tpu as pltpu
from jax.experimental.pallas import tpu_sc as plsc

VOCAB = 1000
EMBED = 64
EPAD = 128
B = 1024
S = 50
S1 = 25
S2 = S - S1

NC = 2
NS = 16
NW = NC * NS
CHUNK = 400


def _make_e_gather(ntok):
    rpw = ntok // NW
    nchunk = rpw // CHUNK

    def body(tbl_hbm, idx_hbm, out_hbm, idx_v, rows0, rows1,
             gsem0, gsem1, wsem0, wsem1):
        wid = lax.axis_index("s") * NC + lax.axis_index("c")
        base = wid * rpw
        pltpu.sync_copy(idx_hbm.at[pl.ds(base, rpw)], idx_v)

        rows = (rows0, rows1)
        gsem = (gsem0, gsem1)
        wsem = (wsem0, wsem1)

        def start_gather(c):
            return pltpu.async_copy(
                tbl_hbm.at[idx_v.at[pl.ds(c * CHUNK, CHUNK)]],
                rows[c % 2], gsem[c % 2])

        def start_write(c):
            return pltpu.async_copy(
                rows[c % 2], out_hbm.at[pl.ds(base + c * CHUNK, CHUNK)],
                wsem[c % 2])

        gathers = [None] * nchunk
        writes = [None] * nchunk
        gathers[0] = start_gather(0)
        for c in range(nchunk):
            gathers[c].wait()
            if c + 1 < nchunk:
                if c >= 1:
                    writes[c - 1].wait()
                gathers[c + 1] = start_gather(c + 1)
            writes[c] = start_write(c)
        writes[nchunk - 1].wait()

    return functools.partial(
        pl.kernel,
        out_type=jax.ShapeDtypeStruct((ntok, EPAD), jnp.float32),
        mesh=plsc.VectorSubcoreMesh(core_axis_name="c", subcore_axis_name="s"),
        scratch_types=[
            pltpu.VMEM((rpw,), jnp.int32),
            pltpu.VMEM((CHUNK, EPAD), jnp.float32),
            pltpu.VMEM((CHUNK, EPAD), jnp.float32),
            pltpu.SemaphoreType.DMA,
            pltpu.SemaphoreType.DMA,
            pltpu.SemaphoreType.DMA,
            pltpu.SemaphoreType.DMA,
        ],
        compiler_params=pltpu.CompilerParams(use_tc_tiling_on_sc=True),
    )(body)


_e_gather_1 = _make_e_gather(S1 * B)
_e_gather_2 = _make_e_gather(S2 * B)


def _proj_body(w_ref, e_ref, o_ref):
    o_ref[0] = lax.dot_general(
        w_ref[...], e_ref[0, :, :EMBED],
        dimension_numbers=(((1,), (1,)), ((), ())),
        preferred_element_type=jnp.float32,
    )


def _proj_body2(prev_ref, w_ref, e_ref, o_ref):
    del prev_ref
    o_ref[0] = lax.dot_general(
        w_ref[...], e_ref[0, :, :EMBED],
        dimension_numbers=(((1,), (1,)), ((), ())),
        preferred_element_type=jnp.float32,
    )


def _project1(head_weight, e_sb):
    return pl.pallas_call(
        _proj_body,
        grid=(S1,),
        in_specs=[
            pl.BlockSpec((VOCAB, EMBED), lambda s: (0, 0)),
            pl.BlockSpec((1, B, EPAD), lambda s: (s, 0, 0)),
        ],
        out_specs=pl.BlockSpec((1, VOCAB, B), lambda s: (s, 0, 0)),
        out_shape=jax.ShapeDtypeStruct((S, VOCAB, B), jnp.float32),
    )(head_weight, e_sb)


def _project2(prev, head_weight, e_sb):
    return pl.pallas_call(
        _proj_body2,
        grid=(S2,),
        in_specs=[
            pl.BlockSpec(memory_space=pl.ANY),
            pl.BlockSpec((VOCAB, EMBED), lambda s: (0, 0)),
            pl.BlockSpec((1, B, EPAD), lambda s: (s, 0, 0)),
        ],
        out_specs=pl.BlockSpec((1, VOCAB, B), lambda s: (s + S1, 0, 0)),
        out_shape=jax.ShapeDtypeStruct((S, VOCAB, B), jnp.float32),
        input_output_aliases={0: 0},
    )(prev, head_weight, e_sb)


def kernel(x, embed_table, head_weight):
    tbl_pad = jnp.zeros((VOCAB, EPAD), jnp.float32).at[:, :EMBED].set(
        embed_table)
    xt1 = x[:, :S1].T.reshape(S1 * B)
    xt2 = x[:, S1:].T.reshape(S2 * B)
    e1 = _e_gather_1(tbl_pad, xt1).reshape(S1, B, EPAD)
    e2 = _e_gather_2(tbl_pad, xt2).reshape(S2, B, EPAD)
    half = _project1(head_weight, e1)
    out_svb = _project2(half, head_weight, e2)
    return out_svb.transpose(2, 0, 1)


# speedup vs baseline: 1.0055x; 1.0055x over previous
import functools

import jax
import jax.numpy as jnp
from jax import lax
from jax.experimental import pallas as pl
from jax.experimental.pallas import tpu as pltpu
from jax.experimental.pallas import tpu_sc as plsc

VOCAB = 1000
EMBED = 64
B = 1024
S = 50

NC = 2
NS = 16
NW = NC * NS
NTOK = B * S
RPW = NTOK // NW


EPAD = 128
CHUNK = 400
NCHUNK = RPW // CHUNK


def _e_gather_body(tbl_hbm, idx_hbm, out_hbm, idx_v, rows0, rows1,
                   gsem0, gsem1, wsem0, wsem1):
    wid = lax.axis_index("s") * NC + lax.axis_index("c")
    base = wid * RPW
    pltpu.sync_copy(idx_hbm.at[pl.ds(base, RPW)], idx_v)

    rows = (rows0, rows1)
    gsem = (gsem0, gsem1)
    wsem = (wsem0, wsem1)

    def start_gather(c):
        return pltpu.async_copy(
            tbl_hbm.at[idx_v.at[pl.ds(c * CHUNK, CHUNK)]],
            rows[c % 2], gsem[c % 2])

    def start_write(c):
        return pltpu.async_copy(
            rows[c % 2], out_hbm.at[pl.ds(base + c * CHUNK, CHUNK)],
            wsem[c % 2])

    gathers = [None] * NCHUNK
    writes = [None] * NCHUNK
    gathers[0] = start_gather(0)
    for c in range(NCHUNK):
        gathers[c].wait()
        if c + 1 < NCHUNK:
            if c >= 1:
                writes[c - 1].wait()
            gathers[c + 1] = start_gather(c + 1)
        writes[c] = start_write(c)
    writes[NCHUNK - 1].wait()


_e_gather = functools.partial(
    pl.kernel,
    out_type=jax.ShapeDtypeStruct((NTOK, EPAD), jnp.float32),
    mesh=plsc.VectorSubcoreMesh(core_axis_name="c", subcore_axis_name="s"),
    scratch_types=[
        pltpu.VMEM((RPW,), jnp.int32),
        pltpu.VMEM((CHUNK, EPAD), jnp.float32),
        pltpu.VMEM((CHUNK, EPAD), jnp.float32),
        pltpu.SemaphoreType.DMA,
        pltpu.SemaphoreType.DMA,
        pltpu.SemaphoreType.DMA,
        pltpu.SemaphoreType.DMA,
    ],
    compiler_params=pltpu.CompilerParams(use_tc_tiling_on_sc=True),
)(_e_gather_body)


def _proj_body(w_ref, e_ref, o_ref):
    o_ref[0] = lax.dot_general(
        w_ref[...], e_ref[0, :, :EMBED],
        dimension_numbers=(((1,), (1,)), ((), ())),
        preferred_element_type=jnp.float32,
    )


def _project(head_weight, e_sb):
    return pl.pallas_call(
        _proj_body,
        grid=(S,),
        in_specs=[
            pl.BlockSpec((VOCAB, EMBED), lambda s: (0, 0)),
            pl.BlockSpec((1, B, EPAD), lambda s: (s, 0, 0)),
        ],
        out_specs=pl.BlockSpec((1, VOCAB, B), lambda s: (s, 0, 0)),
        out_shape=jax.ShapeDtypeStruct((S, VOCAB, B), jnp.float32),
    )(head_weight, e_sb)


def kernel(x, embed_table, head_weight):
    xt = x.T.reshape(NTOK)
    tbl_pad = jnp.zeros((VOCAB, EPAD), jnp.float32).at[:, :EMBED].set(
        embed_table)
    e_flat = _e_gather(tbl_pad, xt)
    e_sb = e_flat.reshape(S, B, EPAD)
    out_svb = _project(head_weight, e_sb)
    return out_svb.transpose(2, 0, 1)


# speedup vs baseline: 1.0071x; 1.0016x over previous
import functools

import jax
import jax.numpy as jnp
from jax import lax
from jax.experimental import pallas as pl
from jax.experimental.pallas import tpu as pltpu
from jax.experimental.pallas import tpu_sc as plsc

VOCAB = 1000
EMBED = 64
B = 1024
S = 50

NC = 2
NS = 16
NW = NC * NS
NTOK = B * S
RPW = NTOK // NW


EPAD = 128
CHUNK = 400
NCHUNK = RPW // CHUNK


def _e_gather_body(tbl_hbm, idx_hbm, out_hbm, idx_v, rows0, rows1,
                   gsem0, gsem1, wsem0, wsem1):
    wid = lax.axis_index("s") * NC + lax.axis_index("c")
    base = wid * RPW
    pltpu.sync_copy(idx_hbm.at[pl.ds(base, RPW)], idx_v)

    rows = (rows0, rows1)
    gsem = (gsem0, gsem1)
    wsem = (wsem0, wsem1)

    def start_gather(c):
        return pltpu.async_copy(
            tbl_hbm.at[idx_v.at[pl.ds(c * CHUNK, CHUNK)]],
            rows[c % 2], gsem[c % 2])

    def start_write(c):
        return pltpu.async_copy(
            rows[c % 2], out_hbm.at[pl.ds(base + c * CHUNK, CHUNK)],
            wsem[c % 2])

    gathers = [None] * NCHUNK
    writes = [None] * NCHUNK
    gathers[0] = start_gather(0)
    for c in range(NCHUNK):
        gathers[c].wait()
        if c + 1 < NCHUNK:
            if c >= 1:
                writes[c - 1].wait()
            gathers[c + 1] = start_gather(c + 1)
        writes[c] = start_write(c)
    writes[NCHUNK - 1].wait()


_e_gather = functools.partial(
    pl.kernel,
    out_type=jax.ShapeDtypeStruct((NTOK, EPAD), jnp.float32),
    mesh=plsc.VectorSubcoreMesh(core_axis_name="c", subcore_axis_name="s",
                                num_cores=NC, num_subcores=NS),
    scratch_types=[
        pltpu.VMEM((RPW,), jnp.int32),
        pltpu.VMEM((CHUNK, EPAD), jnp.float32),
        pltpu.VMEM((CHUNK, EPAD), jnp.float32),
        pltpu.SemaphoreType.DMA,
        pltpu.SemaphoreType.DMA,
        pltpu.SemaphoreType.DMA,
        pltpu.SemaphoreType.DMA,
    ],
    compiler_params=pltpu.CompilerParams(use_tc_tiling_on_sc=True),
)(_e_gather_body)


def _proj_body(w_ref, e_ref, o_ref):
    o_ref[0] = lax.dot_general(
        w_ref[...], e_ref[0, :, :EMBED],
        dimension_numbers=(((1,), (1,)), ((), ())),
        preferred_element_type=jnp.float32,
    )


def _project(head_weight, e_sb):
    return pl.pallas_call(
        _proj_body,
        grid=(S,),
        in_specs=[
            pl.BlockSpec((VOCAB, EMBED), lambda s: (0, 0)),
            pl.BlockSpec((1, B, EPAD), lambda s: (s, 0, 0)),
        ],
        out_specs=pl.BlockSpec((1, VOCAB, B), lambda s: (s, 0, 0)),
        out_shape=jax.ShapeDtypeStruct((S, VOCAB, B), jnp.float32),
    )(head_weight, e_sb)


def kernel(x, embed_table, head_weight):
    xt = x.T.reshape(NTOK)
    tbl_pad = jnp.zeros((VOCAB, EPAD), jnp.float32).at[:, :EMBED].set(
        embed_table)
    e_flat = _e_gather(tbl_pad, xt)
    e_sb = e_flat.reshape(S, B, EPAD)
    out_svb = _project(head_weight, e_sb)
    return out_svb.transpose(2, 0, 1)
